# baseline (speedup 1.0000x reference)
import functools

import jax
import jax.numpy as jnp
from jax import lax
from jax.experimental import pallas as pl
from jax.experimental.pallas import tpu as pltpu
from jax.experimental.pallas import tpu_sc as plsc

_IMIN = -(2 ** 31)


def _matmul_body(x_ref, w_ref, b_ref, z_ref, *, bd):
    j = pl.program_id(0)
    z = jax.lax.dot_general(
        x_ref[...], w_ref[...], (((1,), (1,)), ((), ())),
        preferred_element_type=jnp.float32)
    z_ref[...] = z + b_ref[:, pl.ds(j * bd, bd)]


def _sc_row_prefix24(buf, hist, *, topk, d_dict):
    imin = jnp.int32(_IMIN)
    iota = lax.iota(jnp.int32, 16)
    ones = jnp.ones((16,), jnp.int32)
    zeros16 = jnp.zeros((16,), jnp.int32)

    def splat(v):
        return jnp.broadcast_to(v, (16,)).astype(jnp.int32)

    def level(shift, prefix, prefix_shift, kth):
        def zbody(i, _):
            hist[pl.ds(i * 16, 16)] = zeros16
            return 0
        lax.fori_loop(0, 256, zbody, 0)

        def hbody(g, _):
            v = buf[pl.ds(g * 16, 16)]
            ib = jax.lax.bitcast_convert_type(v, jnp.int32)
            key = jnp.where(ib >= 0, ib, imin - ib - jnp.int32(1))
            if prefix is None:
                b = (key >> jnp.int32(shift)) + jnp.int32(128)
                plsc.addupdate_scatter(hist, [iota * 256 + b], ones)
            else:
                b = (key >> jnp.int32(shift)) & jnp.int32(0xFF)
                m = (key >> jnp.int32(prefix_shift)) == prefix
                plsc.addupdate_scatter(hist, [iota * 256 + b], ones, mask=m)
            return 0
        lax.fori_loop(0, d_dict // 16, hbody, 0)

        def sbody(i, carry):
            s_above, bsel, gsel, found = carry
            bv = jnp.int32(15) - i
            tot = zeros16
            for l in range(16):
                tot = tot + hist[pl.ds(bv * 16 + l * 256, 16)]
            trev = lax.rev(tot, (0,))
            c = plsc.cumsum(trev)
            crossed = (c + s_above) >= kth
            anyc = splat(jnp.sum(crossed.astype(jnp.int32)))
            istar = splat(plsc.all_reduce_ffs(crossed))
            hit = jnp.logical_and(found == 0, anyc > 0)
            b_here = splat(bv * 16 + jnp.int32(15)) - istar
            g_here = s_above + splat(jnp.sum(
                jnp.where(iota < istar, trev, zeros16)))
            bsel = jnp.where(hit, b_here, bsel)
            gsel = jnp.where(hit, g_here, gsel)
            found = jnp.where(hit, splat(1), found)
            s_above = s_above + splat(jnp.sum(tot))
            return (s_above, bsel, gsel, found)

        init = (zeros16, zeros16, zeros16, zeros16)
        _, bsel, gsel, _ = lax.fori_loop(0, 16, sbody, init)
        return bsel, gsel

    b1, g1 = level(24, None, 0, splat(topk))
    top8 = b1 - jnp.int32(128)
    k2 = splat(topk) - g1
    b2, g2 = level(16, top8, 24, k2)
    prefix16 = (top8 << jnp.int32(8)) | b2
    k3 = k2 - g2
    b3, _ = level(8, prefix16, 16, k3)
    prefix24 = (prefix16 << jnp.int32(8)) | b3
    return prefix24 << jnp.int32(8)


def _make_sc_t0(n_tok, d_dict, topk, rows_per_worker, interpret=False):
    mesh = plsc.VectorSubcoreMesh(core_axis_name="c", subcore_axis_name="s")

    @functools.partial(
        pl.kernel, mesh=mesh,
        out_type=jax.ShapeDtypeStruct((n_tok,), jnp.int32),
        scratch_types=[
            pltpu.VMEM((d_dict,), jnp.float32),
            pltpu.VMEM((d_dict,), jnp.float32),
            pltpu.VMEM((16 * 256,), jnp.int32),
            pltpu.VMEM((rows_per_worker,), jnp.int32),
            pltpu.SemaphoreType.DMA,
            pltpu.SemaphoreType.DMA,
        ],
        compiler_params=pltpu.CompilerParams(needs_layout_passes=False),
        interpret=interpret,
    )
    def sc_t0(z_hbm, t0_hbm, buf0, buf1, hist, taubuf, sem0, sem1):
        nc = 2
        wid = lax.axis_index("s") * nc + lax.axis_index("c")
        base = wid * rows_per_worker
        iota = lax.iota(jnp.int32, 16)

        pltpu.make_async_copy(z_hbm.at[base], buf0, sem0).start()
        pltpu.make_async_copy(z_hbm.at[base + 1], buf1, sem1).start()

        def pair(p, _):
            r0 = base + 2 * p
            npairs = rows_per_worker // 2

            pltpu.make_async_copy(z_hbm.at[r0], buf0, sem0).wait()
            t0a = _sc_row_prefix24(buf0, hist, topk=topk, d_dict=d_dict)

            @pl.when(p < npairs - 1)
            def _():
                pltpu.make_async_copy(z_hbm.at[r0 + 2], buf0, sem0).start()

            plsc.store_scatter(
                taubuf, [jnp.broadcast_to(2 * p, (16,)).astype(jnp.int32)],
                t0a, mask=iota == 0)

            pltpu.make_async_copy(z_hbm.at[r0 + 1], buf1, sem1).wait()
            t0b = _sc_row_prefix24(buf1, hist, topk=topk, d_dict=d_dict)

            @pl.when(p < npairs - 1)
            def _():
                pltpu.make_async_copy(z_hbm.at[r0 + 3], buf1, sem1).start()

            plsc.store_scatter(
                taubuf, [jnp.broadcast_to(2 * p + 1, (16,)).astype(jnp.int32)],
                t0b, mask=iota == 0)
            return 0

        lax.fori_loop(0, rows_per_worker // 2, pair, 0)
        pltpu.sync_copy(taubuf, t0_hbm.at[pl.ds(base, rows_per_worker)])

    return sc_t0


def _select_body(z_ref, t0_ref, out_ref, *, topk):
    imin = jnp.int32(_IMIN)
    ib = jax.lax.bitcast_convert_type(z_ref[...], jnp.int32)
    skey = jnp.where(ib >= 0, ib, imin - ib - jnp.int32(1))
    out_ref[...] = jax.lax.bitcast_convert_type(skey, jnp.float32)

    def body(it, t):
        cand = t + jax.lax.shift_left(jnp.int32(1), jnp.int32(7) - it)
        s = jax.lax.bitcast_convert_type(out_ref[...], jnp.int32)
        cnt = jnp.sum((s >= cand).astype(jnp.int32), axis=1, keepdims=True)
        return jnp.where(cnt >= topk, cand, t)

    t = jax.lax.fori_loop(0, 8, body, t0_ref[...])

    s = jax.lax.bitcast_convert_type(out_ref[...], jnp.int32)
    mask = s >= t
    zbits = jnp.where(s >= 0, s, imin - s - jnp.int32(1))
    zrec = jax.lax.bitcast_convert_type(zbits, jnp.float32)
    out_ref[...] = jnp.where(mask, jnp.maximum(zrec, 0.0), 0.0)


def _decode_body(s_ref, w_ref, b_ref, out_ref, *, nk):
    k = pl.program_id(1)
    acc = jax.lax.dot_general(
        s_ref[...], w_ref[...], (((1,), (1,)), ((), ())),
        preferred_element_type=jnp.float32)

    @pl.when(k == 0)
    def _():
        out_ref[...] = acc + b_ref[...]

    @pl.when(k != 0)
    def _():
        out_ref[...] += acc


def _topksae_fwd(x, W_enc, b_enc, W_dec, b_dec, *, topk, tmz, bd, tms, tm2,
                 kd, interpret=False):
    n_tok, d_model = x.shape
    d_dict = W_enc.shape[0]
    niz, nj = n_tok // tmz, d_dict // bd
    b_enc2 = b_enc.reshape(1, d_dict)
    z = pl.pallas_call(
        functools.partial(_matmul_body, bd=bd),
        grid=(nj, niz),
        in_specs=[
            pl.BlockSpec((tmz, d_model), lambda j, i: (i, 0)),
            pl.BlockSpec((bd, d_model), lambda j, i: (j, 0)),
            pl.BlockSpec((1, d_dict), lambda j, i: (0, 0)),
        ],
        out_specs=pl.BlockSpec((tmz, bd), lambda j, i: (i, j)),
        out_shape=jax.ShapeDtypeStruct((n_tok, d_dict), jnp.float32),
        compiler_params=pltpu.CompilerParams(
            dimension_semantics=("parallel", "parallel")),
        interpret=interpret,
    )(x, W_enc, b_enc2)

    t0 = _make_sc_t0(n_tok, d_dict, topk, n_tok // 32, interpret=interpret)(z)

    nis = n_tok // tms
    sparse = pl.pallas_call(
        functools.partial(_select_body, topk=topk),
        grid=(nis,),
        in_specs=[
            pl.BlockSpec((tms, d_dict), lambda i: (i, 0)),
            pl.BlockSpec((tms, 1), lambda i: (i, 0)),
        ],
        out_specs=pl.BlockSpec((tms, d_dict), lambda i: (i, 0)),
        out_shape=jax.ShapeDtypeStruct((n_tok, d_dict), jnp.float32),
        compiler_params=pltpu.CompilerParams(
            dimension_semantics=("parallel",)),
        interpret=interpret,
    )(z, t0.reshape(n_tok, 1))

    ni2, nk = n_tok // tm2, d_dict // kd
    b_dec2 = b_dec.reshape(1, d_model)
    x_hat = pl.pallas_call(
        functools.partial(_decode_body, nk=nk),
        grid=(ni2, nk),
        in_specs=[
            pl.BlockSpec((tm2, kd), lambda i, k: (i, k)),
            pl.BlockSpec((d_model, kd), lambda i, k: (0, k)),
            pl.BlockSpec((1, d_model), lambda i, k: (0, 0)),
        ],
        out_specs=pl.BlockSpec((tm2, d_model), lambda i, k: (i, 0)),
        out_shape=jax.ShapeDtypeStruct((n_tok, d_model), jnp.float32),
        compiler_params=pltpu.CompilerParams(
            dimension_semantics=("parallel", "arbitrary")),
        interpret=interpret,
    )(sparse, W_dec, b_dec2)
    return (x_hat, sparse)


def kernel(x, W_enc, b_enc, W_dec, b_dec):
    return _topksae_fwd(x, W_enc, b_enc, W_dec, b_dec,
                        topk=64, tmz=512, bd=2048, tms=128, tm2=1024, kd=1024)


# speedup vs baseline: 2.5761x; 2.5761x over previous
import functools

import jax
import jax.numpy as jnp
from jax import lax
from jax.experimental import pallas as pl
from jax.experimental.pallas import tpu as pltpu
from jax.experimental.pallas import tpu_sc as plsc

_IMIN = -(2 ** 31)


def _matmul_body(x_ref, w_ref, b_ref, z_ref, *, bd):
    j = pl.program_id(0)
    z = jax.lax.dot_general(
        x_ref[...], w_ref[...], (((1,), (1,)), ((), ())),
        preferred_element_type=jnp.float32)
    z_ref[...] = z + b_ref[:, pl.ds(j * bd, bd)]


def _sc_row_prefix24(buf, hist, *, topk, d_dict):
    imin = jnp.int32(_IMIN)
    iota = lax.iota(jnp.int32, 16)
    ones = jnp.ones((16,), jnp.int32)
    zeros16 = jnp.zeros((16,), jnp.int32)

    def splat(v):
        return jnp.broadcast_to(v, (16,)).astype(jnp.int32)

    def level(shift, prefix, prefix_shift, kth):
        @plsc.parallel_loop(0, 256, unroll=16)
        def _zero(i):
            hist[pl.ds(i * 16, 16)] = zeros16

        @plsc.parallel_loop(0, d_dict // 16, unroll=8)
        def _histp(g):
            v = buf[pl.ds(g * 16, 16)]
            ib = jax.lax.bitcast_convert_type(v, jnp.int32)
            key = jnp.where(ib >= 0, ib, imin - ib - jnp.int32(1))
            if prefix is None:
                b = (key >> jnp.int32(shift)) + jnp.int32(128)
                plsc.addupdate_scatter(hist, [iota * 256 + b], ones)
            else:
                b = (key >> jnp.int32(shift)) & jnp.int32(0xFF)
                m = (key >> jnp.int32(prefix_shift)) == prefix
                plsc.addupdate_scatter(hist, [iota * 256 + b], ones, mask=m)

        def sbody(i, carry):
            s_above, bsel, gsel, found = carry
            bv = jnp.int32(15) - i
            tot = zeros16
            for l in range(16):
                tot = tot + hist[pl.ds(bv * 16 + l * 256, 16)]
            trev = lax.rev(tot, (0,))
            c = plsc.cumsum(trev)
            crossed = (c + s_above) >= kth
            anyc = splat(jnp.sum(crossed.astype(jnp.int32)))
            istar = splat(plsc.all_reduce_ffs(crossed))
            hit = jnp.logical_and(found == 0, anyc > 0)
            b_here = splat(bv * 16 + jnp.int32(15)) - istar
            g_here = s_above + splat(jnp.sum(
                jnp.where(iota < istar, trev, zeros16)))
            bsel = jnp.where(hit, b_here, bsel)
            gsel = jnp.where(hit, g_here, gsel)
            found = jnp.where(hit, splat(1), found)
            s_above = s_above + splat(jnp.sum(tot))
            return (s_above, bsel, gsel, found)

        init = (zeros16, zeros16, zeros16, zeros16)
        _, bsel, gsel, _ = lax.fori_loop(0, 16, sbody, init)
        return bsel, gsel

    b1, g1 = level(24, None, 0, splat(topk))
    top8 = b1 - jnp.int32(128)
    k2 = splat(topk) - g1
    b2, g2 = level(16, top8, 24, k2)
    prefix16 = (top8 << jnp.int32(8)) | b2
    k3 = k2 - g2
    b3, _ = level(8, prefix16, 16, k3)
    prefix24 = (prefix16 << jnp.int32(8)) | b3
    return prefix24 << jnp.int32(8)


def _make_sc_t0(n_tok, d_dict, topk, rows_per_worker, interpret=False):
    mesh = plsc.VectorSubcoreMesh(core_axis_name="c", subcore_axis_name="s")

    @functools.partial(
        pl.kernel, mesh=mesh,
        out_type=jax.ShapeDtypeStruct((n_tok,), jnp.int32),
        scratch_types=[
            pltpu.VMEM((d_dict,), jnp.float32),
            pltpu.VMEM((d_dict,), jnp.float32),
            pltpu.VMEM((16 * 256,), jnp.int32),
            pltpu.VMEM((rows_per_worker,), jnp.int32),
            pltpu.SemaphoreType.DMA,
            pltpu.SemaphoreType.DMA,
        ],
        compiler_params=pltpu.CompilerParams(needs_layout_passes=False),
        interpret=interpret,
    )
    def sc_t0(z_hbm, t0_hbm, buf0, buf1, hist, taubuf, sem0, sem1):
        nc = 2
        wid = lax.axis_index("s") * nc + lax.axis_index("c")
        base = wid * rows_per_worker
        iota = lax.iota(jnp.int32, 16)

        pltpu.make_async_copy(z_hbm.at[base], buf0, sem0).start()
        pltpu.make_async_copy(z_hbm.at[base + 1], buf1, sem1).start()

        def pair(p, _):
            r0 = base + 2 * p
            npairs = rows_per_worker // 2

            pltpu.make_async_copy(z_hbm.at[r0], buf0, sem0).wait()
            t0a = _sc_row_prefix24(buf0, hist, topk=topk, d_dict=d_dict)

            @pl.when(p < npairs - 1)
            def _():
                pltpu.make_async_copy(z_hbm.at[r0 + 2], buf0, sem0).start()

            plsc.store_scatter(
                taubuf, [jnp.broadcast_to(2 * p, (16,)).astype(jnp.int32)],
                t0a, mask=iota == 0)

            pltpu.make_async_copy(z_hbm.at[r0 + 1], buf1, sem1).wait()
            t0b = _sc_row_prefix24(buf1, hist, topk=topk, d_dict=d_dict)

            @pl.when(p < npairs - 1)
            def _():
                pltpu.make_async_copy(z_hbm.at[r0 + 3], buf1, sem1).start()

            plsc.store_scatter(
                taubuf, [jnp.broadcast_to(2 * p + 1, (16,)).astype(jnp.int32)],
                t0b, mask=iota == 0)
            return 0

        lax.fori_loop(0, rows_per_worker // 2, pair, 0)
        pltpu.sync_copy(taubuf, t0_hbm.at[pl.ds(base, rows_per_worker)])

    return sc_t0


def _select_body(z_ref, t0_ref, out_ref, *, topk):
    imin = jnp.int32(_IMIN)
    ib = jax.lax.bitcast_convert_type(z_ref[...], jnp.int32)
    skey = jnp.where(ib >= 0, ib, imin - ib - jnp.int32(1))
    out_ref[...] = jax.lax.bitcast_convert_type(skey, jnp.float32)

    def body(it, t):
        cand = t + jax.lax.shift_left(jnp.int32(1), jnp.int32(7) - it)
        s = jax.lax.bitcast_convert_type(out_ref[...], jnp.int32)
        cnt = jnp.sum((s >= cand).astype(jnp.int32), axis=1, keepdims=True)
        return jnp.where(cnt >= topk, cand, t)

    t = jax.lax.fori_loop(0, 8, body, t0_ref[...])

    s = jax.lax.bitcast_convert_type(out_ref[...], jnp.int32)
    mask = s >= t
    zbits = jnp.where(s >= 0, s, imin - s - jnp.int32(1))
    zrec = jax.lax.bitcast_convert_type(zbits, jnp.float32)
    out_ref[...] = jnp.where(mask, jnp.maximum(zrec, 0.0), 0.0)


def _decode_body(s_ref, w_ref, b_ref, out_ref, *, nk):
    k = pl.program_id(1)
    acc = jax.lax.dot_general(
        s_ref[...], w_ref[...], (((1,), (1,)), ((), ())),
        preferred_element_type=jnp.float32)

    @pl.when(k == 0)
    def _():
        out_ref[...] = acc + b_ref[...]

    @pl.when(k != 0)
    def _():
        out_ref[...] += acc


def _topksae_fwd(x, W_enc, b_enc, W_dec, b_dec, *, topk, tmz, bd, tms, tm2,
                 kd, interpret=False):
    n_tok, d_model = x.shape
    d_dict = W_enc.shape[0]
    niz, nj = n_tok // tmz, d_dict // bd
    b_enc2 = b_enc.reshape(1, d_dict)
    z = pl.pallas_call(
        functools.partial(_matmul_body, bd=bd),
        grid=(nj, niz),
        in_specs=[
            pl.BlockSpec((tmz, d_model), lambda j, i: (i, 0)),
            pl.BlockSpec((bd, d_model), lambda j, i: (j, 0)),
            pl.BlockSpec((1, d_dict), lambda j, i: (0, 0)),
        ],
        out_specs=pl.BlockSpec((tmz, bd), lambda j, i: (i, j)),
        out_shape=jax.ShapeDtypeStruct((n_tok, d_dict), jnp.float32),
        compiler_params=pltpu.CompilerParams(
            dimension_semantics=("parallel", "parallel")),
        interpret=interpret,
    )(x, W_enc, b_enc2)

    t0 = _make_sc_t0(n_tok, d_dict, topk, n_tok // 32, interpret=interpret)(z)

    nis = n_tok // tms
    sparse = pl.pallas_call(
        functools.partial(_select_body, topk=topk),
        grid=(nis,),
        in_specs=[
            pl.BlockSpec((tms, d_dict), lambda i: (i, 0)),
            pl.BlockSpec((tms, 1), lambda i: (i, 0)),
        ],
        out_specs=pl.BlockSpec((tms, d_dict), lambda i: (i, 0)),
        out_shape=jax.ShapeDtypeStruct((n_tok, d_dict), jnp.float32),
        compiler_params=pltpu.CompilerParams(
            dimension_semantics=("parallel",)),
        interpret=interpret,
    )(z, t0.reshape(n_tok, 1))

    ni2, nk = n_tok // tm2, d_dict // kd
    b_dec2 = b_dec.reshape(1, d_model)
    x_hat = pl.pallas_call(
        functools.partial(_decode_body, nk=nk),
        grid=(ni2, nk),
        in_specs=[
            pl.BlockSpec((tm2, kd), lambda i, k: (i, k)),
            pl.BlockSpec((d_model, kd), lambda i, k: (0, k)),
            pl.BlockSpec((1, d_model), lambda i, k: (0, 0)),
        ],
        out_specs=pl.BlockSpec((tm2, d_model), lambda i, k: (i, 0)),
        out_shape=jax.ShapeDtypeStruct((n_tok, d_model), jnp.float32),
        compiler_params=pltpu.CompilerParams(
            dimension_semantics=("parallel", "arbitrary")),
        interpret=interpret,
    )(sparse, W_dec, b_dec2)
    return (x_hat, sparse)


def kernel(x, W_enc, b_enc, W_dec, b_dec):
    return _topksae_fwd(x, W_enc, b_enc, W_dec, b_dec,
                        topk=64, tmz=512, bd=2048, tms=128, tm2=1024, kd=1024)


# speedup vs baseline: 2.6985x; 1.0475x over previous
import functools

import jax
import jax.numpy as jnp
from jax import lax
from jax.experimental import pallas as pl
from jax.experimental.pallas import tpu as pltpu
from jax.experimental.pallas import tpu_sc as plsc

_IMIN = -(2 ** 31)


def _matmul_body(x_ref, w_ref, b_ref, z_ref, *, bd):
    j = pl.program_id(0)
    z = jax.lax.dot_general(
        x_ref[...], w_ref[...], (((1,), (1,)), ((), ())),
        preferred_element_type=jnp.float32)
    z_ref[...] = z + b_ref[:, pl.ds(j * bd, bd)]


def _sc_row_tau(buf, kbuf, hist, *, topk, d_dict):
    imin = jnp.int32(_IMIN)
    iota = lax.iota(jnp.int32, 16)
    ones = jnp.ones((16,), jnp.int32)
    zeros16 = jnp.zeros((16,), jnp.int32)

    def splat(v):
        return jnp.broadcast_to(v, (16,)).astype(jnp.int32)

    @plsc.parallel_loop(0, d_dict // 16, unroll=8)
    def _mapk(g):
        v = buf[pl.ds(g * 16, 16)]
        ib = jax.lax.bitcast_convert_type(v, jnp.int32)
        kbuf[pl.ds(g * 16, 16)] = jnp.where(
            ib >= 0, ib, imin - ib - jnp.int32(1))

    def level(shift, prefix, prefix_shift, kth):
        @plsc.parallel_loop(0, 256, unroll=16)
        def _zero(i):
            hist[pl.ds(i * 16, 16)] = zeros16

        @plsc.parallel_loop(0, d_dict // 16, unroll=8)
        def _histp(g):
            key = kbuf[pl.ds(g * 16, 16)]
            if prefix is None:
                b = (key >> jnp.int32(shift)) + jnp.int32(128)
                plsc.addupdate_scatter(hist, [iota * 256 + b], ones)
            else:
                b = (key >> jnp.int32(shift)) & jnp.int32(0xFF)
                m = (key >> jnp.int32(prefix_shift)) == prefix
                plsc.addupdate_scatter(hist, [iota * 256 + b], ones, mask=m)

        def sbody(i, carry):
            s_above, bsel, gsel, found = carry
            bv = jnp.int32(15) - i
            tot = zeros16
            for l in range(16):
                tot = tot + hist[pl.ds(bv * 16 + l * 256, 16)]
            trev = lax.rev(tot, (0,))
            c = plsc.cumsum(trev)
            crossed = (c + s_above) >= kth
            anyc = splat(jnp.sum(crossed.astype(jnp.int32)))
            istar = splat(plsc.all_reduce_ffs(crossed))
            hit = jnp.logical_and(found == 0, anyc > 0)
            b_here = splat(bv * 16 + jnp.int32(15)) - istar
            g_here = s_above + splat(jnp.sum(
                jnp.where(iota < istar, trev, zeros16)))
            bsel = jnp.where(hit, b_here, bsel)
            gsel = jnp.where(hit, g_here, gsel)
            found = jnp.where(hit, splat(1), found)
            s_above = s_above + splat(jnp.sum(tot))
            return (s_above, bsel, gsel, found)

        init = (zeros16, zeros16, zeros16, zeros16)
        _, bsel, gsel, _ = lax.fori_loop(0, 16, sbody, init)
        return bsel, gsel

    b1, g1 = level(24, None, 0, splat(topk))
    top8 = b1 - jnp.int32(128)
    k2 = splat(topk) - g1
    b2, g2 = level(16, top8, 24, k2)
    prefix16 = (top8 << jnp.int32(8)) | b2
    k3 = k2 - g2
    b3, g3 = level(8, prefix16, 16, k3)
    prefix24 = (prefix16 << jnp.int32(8)) | b3
    k4 = k3 - g3
    b4, _ = level(0, prefix24, 8, k4)
    return (prefix24 << jnp.int32(8)) | b4


def _make_sc_t0(n_tok, d_dict, topk, rows_per_worker, interpret=False):
    mesh = plsc.VectorSubcoreMesh(core_axis_name="c", subcore_axis_name="s")

    @functools.partial(
        pl.kernel, mesh=mesh,
        out_type=jax.ShapeDtypeStruct((n_tok,), jnp.int32),
        scratch_types=[
            pltpu.VMEM((d_dict,), jnp.float32),
            pltpu.VMEM((d_dict,), jnp.float32),
            pltpu.VMEM((d_dict,), jnp.int32),
            pltpu.VMEM((16 * 256,), jnp.int32),
            pltpu.VMEM((rows_per_worker,), jnp.int32),
            pltpu.SemaphoreType.DMA,
            pltpu.SemaphoreType.DMA,
        ],
        compiler_params=pltpu.CompilerParams(needs_layout_passes=False),
        interpret=interpret,
    )
    def sc_t0(z_hbm, t0_hbm, buf0, buf1, kbuf, hist, taubuf, sem0, sem1):
        nc = 2
        wid = lax.axis_index("s") * nc + lax.axis_index("c")
        base = wid * rows_per_worker
        iota = lax.iota(jnp.int32, 16)

        pltpu.make_async_copy(z_hbm.at[base], buf0, sem0).start()
        pltpu.make_async_copy(z_hbm.at[base + 1], buf1, sem1).start()

        def pair(p, _):
            r0 = base + 2 * p
            npairs = rows_per_worker // 2

            pltpu.make_async_copy(z_hbm.at[r0], buf0, sem0).wait()
            t0a = _sc_row_tau(buf0, kbuf, hist, topk=topk, d_dict=d_dict)

            @pl.when(p < npairs - 1)
            def _():
                pltpu.make_async_copy(z_hbm.at[r0 + 2], buf0, sem0).start()

            plsc.store_scatter(
                taubuf, [jnp.broadcast_to(2 * p, (16,)).astype(jnp.int32)],
                t0a, mask=iota == 0)

            pltpu.make_async_copy(z_hbm.at[r0 + 1], buf1, sem1).wait()
            t0b = _sc_row_tau(buf1, kbuf, hist, topk=topk, d_dict=d_dict)

            @pl.when(p < npairs - 1)
            def _():
                pltpu.make_async_copy(z_hbm.at[r0 + 3], buf1, sem1).start()

            plsc.store_scatter(
                taubuf, [jnp.broadcast_to(2 * p + 1, (16,)).astype(jnp.int32)],
                t0b, mask=iota == 0)
            return 0

        lax.fori_loop(0, rows_per_worker // 2, pair, 0)
        pltpu.sync_copy(taubuf, t0_hbm.at[pl.ds(base, rows_per_worker)])

    return sc_t0


def _decode_fused_body(z_ref, t_ref, w_ref, b_ref, out_ref, sp_ref, *, nk):
    k = pl.program_id(1)
    imin = jnp.int32(_IMIN)
    ib = jax.lax.bitcast_convert_type(z_ref[...], jnp.int32)
    skey = jnp.where(ib >= 0, ib, imin - ib - jnp.int32(1))
    mask = skey >= t_ref[...]
    sp = jnp.where(mask, jnp.maximum(z_ref[...], 0.0), 0.0)
    sp_ref[...] = sp
    acc = jax.lax.dot_general(
        sp, w_ref[...], (((1,), (1,)), ((), ())),
        preferred_element_type=jnp.float32)

    @pl.when(k == 0)
    def _():
        out_ref[...] = acc + b_ref[...]

    @pl.when(k != 0)
    def _():
        out_ref[...] += acc


def _topksae_fwd(x, W_enc, b_enc, W_dec, b_dec, *, topk, tmz, bd, tm2, kd,
                 interpret=False):
    n_tok, d_model = x.shape
    d_dict = W_enc.shape[0]
    niz, nj = n_tok // tmz, d_dict // bd
    b_enc2 = b_enc.reshape(1, d_dict)
    z = pl.pallas_call(
        functools.partial(_matmul_body, bd=bd),
        grid=(nj, niz),
        in_specs=[
            pl.BlockSpec((tmz, d_model), lambda j, i: (i, 0)),
            pl.BlockSpec((bd, d_model), lambda j, i: (j, 0)),
            pl.BlockSpec((1, d_dict), lambda j, i: (0, 0)),
        ],
        out_specs=pl.BlockSpec((tmz, bd), lambda j, i: (i, j)),
        out_shape=jax.ShapeDtypeStruct((n_tok, d_dict), jnp.float32),
        compiler_params=pltpu.CompilerParams(
            dimension_semantics=("parallel", "parallel")),
        interpret=interpret,
    )(x, W_enc, b_enc2)

    tau = _make_sc_t0(n_tok, d_dict, topk, n_tok // 32,
                      interpret=interpret)(z)

    ni2, nk = n_tok // tm2, d_dict // kd
    b_dec2 = b_dec.reshape(1, d_model)
    x_hat, sparse = pl.pallas_call(
        functools.partial(_decode_fused_body, nk=nk),
        grid=(ni2, nk),
        in_specs=[
            pl.BlockSpec((tm2, kd), lambda i, k: (i, k)),
            pl.BlockSpec((tm2, 1), lambda i, k: (i, 0)),
            pl.BlockSpec((d_model, kd), lambda i, k: (0, k)),
            pl.BlockSpec((1, d_model), lambda i, k: (0, 0)),
        ],
        out_specs=[
            pl.BlockSpec((tm2, d_model), lambda i, k: (i, 0)),
            pl.BlockSpec((tm2, kd), lambda i, k: (i, k)),
        ],
        out_shape=[
            jax.ShapeDtypeStruct((n_tok, d_model), jnp.float32),
            jax.ShapeDtypeStruct((n_tok, d_dict), jnp.float32),
        ],
        compiler_params=pltpu.CompilerParams(
            dimension_semantics=("parallel", "arbitrary")),
        interpret=interpret,
    )(z, tau.reshape(n_tok, 1), W_dec, b_dec2)
    return (x_hat, sparse)


def kernel(x, W_enc, b_enc, W_dec, b_dec):
    return _topksae_fwd(x, W_enc, b_enc, W_dec, b_dec,
                        topk=64, tmz=512, bd=2048, tm2=512, kd=1024)


# speedup vs baseline: 2.7692x; 1.0262x over previous
import functools

import jax
import jax.numpy as jnp
from jax import lax
from jax.experimental import pallas as pl
from jax.experimental.pallas import tpu as pltpu
from jax.experimental.pallas import tpu_sc as plsc

_IMIN = -(2 ** 31)


def _matmul_body(x_ref, w_ref, b_ref, z_ref, *, bd):
    j = pl.program_id(0)
    z = jax.lax.dot_general(
        x_ref[...], w_ref[...], (((1,), (1,)), ((), ())),
        preferred_element_type=jnp.float32)
    z_ref[...] = z + b_ref[:, pl.ds(j * bd, bd)]


def _sc_row_tau(buf, kbuf, hist, *, topk, d_dict):
    imin = jnp.int32(_IMIN)
    iota = lax.iota(jnp.int32, 16)
    ones = jnp.ones((16,), jnp.int32)
    zeros16 = jnp.zeros((16,), jnp.int32)

    def splat(v):
        return jnp.broadcast_to(v, (16,)).astype(jnp.int32)

    def level(shift, prefix, prefix_shift, kth):
        @plsc.parallel_loop(0, 256, unroll=16)
        def _zero(i):
            hist[pl.ds(i * 16, 16)] = zeros16

        @plsc.parallel_loop(0, d_dict // 16, unroll=16)
        def _histp(g):
            if prefix is None:
                v = buf[pl.ds(g * 16, 16)]
                ib = jax.lax.bitcast_convert_type(v, jnp.int32)
                key = jnp.where(ib >= 0, ib, imin - ib - jnp.int32(1))
                kbuf[pl.ds(g * 16, 16)] = key
                b = (key >> jnp.int32(shift)) + jnp.int32(128)
                plsc.addupdate_scatter(hist, [iota * 256 + b], ones)
            else:
                key = kbuf[pl.ds(g * 16, 16)]
                b = (key >> jnp.int32(shift)) & jnp.int32(0xFF)
                m = (key >> jnp.int32(prefix_shift)) == prefix
                plsc.addupdate_scatter(hist, [iota * 256 + b], ones, mask=m)

        def sbody(i, carry):
            s_above, bsel, gsel, found = carry
            bv = jnp.int32(15) - i
            tot = zeros16
            for l in range(16):
                tot = tot + hist[pl.ds(bv * 16 + l * 256, 16)]
            trev = lax.rev(tot, (0,))
            c = plsc.cumsum(trev)
            crossed = (c + s_above) >= kth
            anyc = splat(jnp.sum(crossed.astype(jnp.int32)))
            istar = splat(plsc.all_reduce_ffs(crossed))
            hit = jnp.logical_and(found == 0, anyc > 0)
            b_here = splat(bv * 16 + jnp.int32(15)) - istar
            g_here = s_above + splat(jnp.sum(
                jnp.where(iota < istar, trev, zeros16)))
            bsel = jnp.where(hit, b_here, bsel)
            gsel = jnp.where(hit, g_here, gsel)
            found = jnp.where(hit, splat(1), found)
            s_above = s_above + splat(jnp.sum(tot))
            return (s_above, bsel, gsel, found)

        init = (zeros16, zeros16, zeros16, zeros16)
        _, bsel, gsel, _ = lax.fori_loop(0, 16, sbody, init)
        return bsel, gsel

    b1, g1 = level(24, None, 0, splat(topk))
    top8 = b1 - jnp.int32(128)
    k2 = splat(topk) - g1
    b2, g2 = level(16, top8, 24, k2)
    prefix16 = (top8 << jnp.int32(8)) | b2
    k3 = k2 - g2
    b3, g3 = level(8, prefix16, 16, k3)
    prefix24 = (prefix16 << jnp.int32(8)) | b3
    k4 = k3 - g3
    b4, _ = level(0, prefix24, 8, k4)
    return (prefix24 << jnp.int32(8)) | b4


def _make_sc_t0(n_tok, d_dict, topk, rows_per_worker, interpret=False):
    mesh = plsc.VectorSubcoreMesh(core_axis_name="c", subcore_axis_name="s")

    @functools.partial(
        pl.kernel, mesh=mesh,
        out_type=jax.ShapeDtypeStruct((n_tok,), jnp.int32),
        scratch_types=[
            pltpu.VMEM((d_dict,), jnp.float32),
            pltpu.VMEM((d_dict,), jnp.float32),
            pltpu.VMEM((d_dict,), jnp.int32),
            pltpu.VMEM((16 * 256,), jnp.int32),
            pltpu.VMEM((rows_per_worker,), jnp.int32),
            pltpu.SemaphoreType.DMA,
            pltpu.SemaphoreType.DMA,
        ],
        compiler_params=pltpu.CompilerParams(needs_layout_passes=False),
        interpret=interpret,
    )
    def sc_t0(z_hbm, t0_hbm, buf0, buf1, kbuf, hist, taubuf, sem0, sem1):
        nc = 2
        wid = lax.axis_index("s") * nc + lax.axis_index("c")
        base = wid * rows_per_worker
        iota = lax.iota(jnp.int32, 16)

        pltpu.make_async_copy(z_hbm.at[base], buf0, sem0).start()
        pltpu.make_async_copy(z_hbm.at[base + 1], buf1, sem1).start()

        def pair(p, _):
            r0 = base + 2 * p
            npairs = rows_per_worker // 2

            pltpu.make_async_copy(z_hbm.at[r0], buf0, sem0).wait()
            t0a = _sc_row_tau(buf0, kbuf, hist, topk=topk, d_dict=d_dict)

            @pl.when(p < npairs - 1)
            def _():
                pltpu.make_async_copy(z_hbm.at[r0 + 2], buf0, sem0).start()

            plsc.store_scatter(
                taubuf, [jnp.broadcast_to(2 * p, (16,)).astype(jnp.int32)],
                t0a, mask=iota == 0)

            pltpu.make_async_copy(z_hbm.at[r0 + 1], buf1, sem1).wait()
            t0b = _sc_row_tau(buf1, kbuf, hist, topk=topk, d_dict=d_dict)

            @pl.when(p < npairs - 1)
            def _():
                pltpu.make_async_copy(z_hbm.at[r0 + 3], buf1, sem1).start()

            plsc.store_scatter(
                taubuf, [jnp.broadcast_to(2 * p + 1, (16,)).astype(jnp.int32)],
                t0b, mask=iota == 0)
            return 0

        lax.fori_loop(0, rows_per_worker // 2, pair, 0)
        pltpu.sync_copy(taubuf, t0_hbm.at[pl.ds(base, rows_per_worker)])

    return sc_t0


def _decode_fused_body(z_ref, t_ref, w_ref, b_ref, out_ref, sp_ref, *, nk):
    k = pl.program_id(1)
    imin = jnp.int32(_IMIN)
    ib = jax.lax.bitcast_convert_type(z_ref[...], jnp.int32)
    skey = jnp.where(ib >= 0, ib, imin - ib - jnp.int32(1))
    mask = skey >= t_ref[...]
    sp = jnp.where(mask, jnp.maximum(z_ref[...], 0.0), 0.0)
    sp_ref[...] = sp
    acc = jax.lax.dot_general(
        sp, w_ref[...], (((1,), (1,)), ((), ())),
        preferred_element_type=jnp.float32)

    @pl.when(k == 0)
    def _():
        out_ref[...] = acc + b_ref[...]

    @pl.when(k != 0)
    def _():
        out_ref[...] += acc


def _topksae_fwd(x, W_enc, b_enc, W_dec, b_dec, *, topk, tmz, bd, tm2, kd,
                 interpret=False):
    n_tok, d_model = x.shape
    d_dict = W_enc.shape[0]
    niz, nj = n_tok // tmz, d_dict // bd
    b_enc2 = b_enc.reshape(1, d_dict)
    z = pl.pallas_call(
        functools.partial(_matmul_body, bd=bd),
        grid=(nj, niz),
        in_specs=[
            pl.BlockSpec((tmz, d_model), lambda j, i: (i, 0)),
            pl.BlockSpec((bd, d_model), lambda j, i: (j, 0)),
            pl.BlockSpec((1, d_dict), lambda j, i: (0, 0)),
        ],
        out_specs=pl.BlockSpec((tmz, bd), lambda j, i: (i, j)),
        out_shape=jax.ShapeDtypeStruct((n_tok, d_dict), jnp.float32),
        compiler_params=pltpu.CompilerParams(
            dimension_semantics=("parallel", "parallel")),
        interpret=interpret,
    )(x, W_enc, b_enc2)

    tau = _make_sc_t0(n_tok, d_dict, topk, n_tok // 32,
                      interpret=interpret)(z)

    ni2, nk = n_tok // tm2, d_dict // kd
    b_dec2 = b_dec.reshape(1, d_model)
    x_hat, sparse = pl.pallas_call(
        functools.partial(_decode_fused_body, nk=nk),
        grid=(ni2, nk),
        in_specs=[
            pl.BlockSpec((tm2, kd), lambda i, k: (i, k)),
            pl.BlockSpec((tm2, 1), lambda i, k: (i, 0)),
            pl.BlockSpec((d_model, kd), lambda i, k: (0, k)),
            pl.BlockSpec((1, d_model), lambda i, k: (0, 0)),
        ],
        out_specs=[
            pl.BlockSpec((tm2, d_model), lambda i, k: (i, 0)),
            pl.BlockSpec((tm2, kd), lambda i, k: (i, k)),
        ],
        out_shape=[
            jax.ShapeDtypeStruct((n_tok, d_model), jnp.float32),
            jax.ShapeDtypeStruct((n_tok, d_dict), jnp.float32),
        ],
        compiler_params=pltpu.CompilerParams(
            dimension_semantics=("parallel", "arbitrary")),
        interpret=interpret,
    )(z, tau.reshape(n_tok, 1), W_dec, b_dec2)
    return (x_hat, sparse)


def kernel(x, W_enc, b_enc, W_dec, b_dec):
    return _topksae_fwd(x, W_enc, b_enc, W_dec, b_dec,
                        topk=64, tmz=512, bd=2048, tm2=512, kd=1024)


# speedup vs baseline: 2.7729x; 1.0013x over previous
import functools

import jax
import jax.numpy as jnp
from jax import lax
from jax.experimental import pallas as pl
from jax.experimental.pallas import tpu as pltpu
from jax.experimental.pallas import tpu_sc as plsc

_IMIN = -(2 ** 31)


def _matmul_body(x_ref, w_ref, b_ref, z_ref, *, bd):
    j = pl.program_id(0)
    z = jax.lax.dot_general(
        x_ref[...], w_ref[...], (((1,), (1,)), ((), ())),
        preferred_element_type=jnp.float32)
    z_ref[...] = z + b_ref[:, pl.ds(j * bd, bd)]


def _sc_row_tau(buf, kbuf, hist, *, topk, d_dict):
    imin = jnp.int32(_IMIN)
    iota = lax.iota(jnp.int32, 16)
    ones = jnp.ones((16,), jnp.int32)
    zeros16 = jnp.zeros((16,), jnp.int32)

    def splat(v):
        return jnp.broadcast_to(v, (16,)).astype(jnp.int32)

    def level(shift, prefix, prefix_shift, kth):
        @plsc.parallel_loop(0, 256, unroll=16)
        def _zero(i):
            hist[pl.ds(i * 16, 16)] = zeros16

        @plsc.parallel_loop(0, d_dict // 16, unroll=16)
        def _histp(g):
            if prefix is None:
                v = buf[pl.ds(g * 16, 16)]
                ib = jax.lax.bitcast_convert_type(v, jnp.int32)
                key = jnp.where(ib >= 0, ib, imin - ib - jnp.int32(1))
                kbuf[pl.ds(g * 16, 16)] = key
                b = (key >> jnp.int32(shift)) + jnp.int32(128)
                plsc.addupdate_scatter(hist, [iota * 256 + b], ones)
            else:
                key = kbuf[pl.ds(g * 16, 16)]
                b = (key >> jnp.int32(shift)) & jnp.int32(0xFF)
                m = (key >> jnp.int32(prefix_shift)) == prefix
                plsc.addupdate_scatter(hist, [iota * 256 + b], ones, mask=m)

        fifteen = jnp.full((16,), 15, jnp.int32)

        def take16(v, idx):
            dnums = lax.GatherDimensionNumbers(
                offset_dims=(), collapsed_slice_dims=(0,),
                start_index_map=(0,))
            return lax.gather(
                v, idx[:, None], dnums, (1,),
                mode=lax.GatherScatterMode.PROMISE_IN_BOUNDS)

        def sbody(i, carry):
            s_above, bsel, gsel, found = carry
            bv = jnp.int32(15) - i
            tot = zeros16
            for l in range(16):
                tot = tot + hist[pl.ds(bv * 16 + l * 256, 16)]
            trev = lax.rev(tot, (0,))
            c = plsc.cumsum(trev)
            tot_all = take16(c, fifteen)
            crossed = (c + s_above) >= kth
            anyc = (tot_all + s_above) >= kth
            istar = splat(plsc.all_reduce_ffs(crossed))
            istar = jnp.where(anyc, istar, zeros16)
            hit = jnp.logical_and(found == 0, anyc)
            b_here = splat(bv * 16 + jnp.int32(15)) - istar
            g_here = s_above + take16(c, istar) - take16(trev, istar)
            bsel = jnp.where(hit, b_here, bsel)
            gsel = jnp.where(hit, g_here, gsel)
            found = jnp.where(hit, splat(1), found)
            s_above = s_above + tot_all
            return (s_above, bsel, gsel, found)

        init = (zeros16, zeros16, zeros16, zeros16)
        _, bsel, gsel, _ = lax.fori_loop(0, 16, sbody, init)
        return bsel, gsel

    b1, g1 = level(24, None, 0, splat(topk))
    top8 = b1 - jnp.int32(128)
    k2 = splat(topk) - g1
    b2, g2 = level(16, top8, 24, k2)
    prefix16 = (top8 << jnp.int32(8)) | b2
    k3 = k2 - g2
    b3, g3 = level(8, prefix16, 16, k3)
    prefix24 = (prefix16 << jnp.int32(8)) | b3
    k4 = k3 - g3
    b4, _ = level(0, prefix24, 8, k4)
    return (prefix24 << jnp.int32(8)) | b4


def _make_sc_t0(n_tok, d_dict, topk, rows_per_worker, interpret=False):
    mesh = plsc.VectorSubcoreMesh(core_axis_name="c", subcore_axis_name="s")

    @functools.partial(
        pl.kernel, mesh=mesh,
        out_type=jax.ShapeDtypeStruct((n_tok,), jnp.int32),
        scratch_types=[
            pltpu.VMEM((d_dict,), jnp.float32),
            pltpu.VMEM((d_dict,), jnp.float32),
            pltpu.VMEM((d_dict,), jnp.int32),
            pltpu.VMEM((16 * 256,), jnp.int32),
            pltpu.VMEM((rows_per_worker,), jnp.int32),
            pltpu.SemaphoreType.DMA,
            pltpu.SemaphoreType.DMA,
        ],
        compiler_params=pltpu.CompilerParams(needs_layout_passes=False),
        interpret=interpret,
    )
    def sc_t0(z_hbm, t0_hbm, buf0, buf1, kbuf, hist, taubuf, sem0, sem1):
        nc = 2
        wid = lax.axis_index("s") * nc + lax.axis_index("c")
        base = wid * rows_per_worker
        iota = lax.iota(jnp.int32, 16)

        pltpu.make_async_copy(z_hbm.at[base], buf0, sem0).start()
        pltpu.make_async_copy(z_hbm.at[base + 1], buf1, sem1).start()

        def pair(p, _):
            r0 = base + 2 * p
            npairs = rows_per_worker // 2

            pltpu.make_async_copy(z_hbm.at[r0], buf0, sem0).wait()
            t0a = _sc_row_tau(buf0, kbuf, hist, topk=topk, d_dict=d_dict)

            @pl.when(p < npairs - 1)
            def _():
                pltpu.make_async_copy(z_hbm.at[r0 + 2], buf0, sem0).start()

            plsc.store_scatter(
                taubuf, [jnp.broadcast_to(2 * p, (16,)).astype(jnp.int32)],
                t0a, mask=iota == 0)

            pltpu.make_async_copy(z_hbm.at[r0 + 1], buf1, sem1).wait()
            t0b = _sc_row_tau(buf1, kbuf, hist, topk=topk, d_dict=d_dict)

            @pl.when(p < npairs - 1)
            def _():
                pltpu.make_async_copy(z_hbm.at[r0 + 3], buf1, sem1).start()

            plsc.store_scatter(
                taubuf, [jnp.broadcast_to(2 * p + 1, (16,)).astype(jnp.int32)],
                t0b, mask=iota == 0)
            return 0

        lax.fori_loop(0, rows_per_worker // 2, pair, 0)
        pltpu.sync_copy(taubuf, t0_hbm.at[pl.ds(base, rows_per_worker)])

    return sc_t0


def _decode_fused_body(z_ref, t_ref, w_ref, b_ref, out_ref, sp_ref, *, nk):
    k = pl.program_id(1)
    imin = jnp.int32(_IMIN)
    ib = jax.lax.bitcast_convert_type(z_ref[...], jnp.int32)
    skey = jnp.where(ib >= 0, ib, imin - ib - jnp.int32(1))
    mask = skey >= t_ref[...]
    sp = jnp.where(mask, jnp.maximum(z_ref[...], 0.0), 0.0)
    sp_ref[...] = sp
    acc = jax.lax.dot_general(
        sp, w_ref[...], (((1,), (1,)), ((), ())),
        preferred_element_type=jnp.float32)

    @pl.when(k == 0)
    def _():
        out_ref[...] = acc + b_ref[...]

    @pl.when(k != 0)
    def _():
        out_ref[...] += acc


def _topksae_fwd(x, W_enc, b_enc, W_dec, b_dec, *, topk, tmz, bd, tm2, kd,
                 interpret=False):
    n_tok, d_model = x.shape
    d_dict = W_enc.shape[0]
    niz, nj = n_tok // tmz, d_dict // bd
    b_enc2 = b_enc.reshape(1, d_dict)
    z = pl.pallas_call(
        functools.partial(_matmul_body, bd=bd),
        grid=(nj, niz),
        in_specs=[
            pl.BlockSpec((tmz, d_model), lambda j, i: (i, 0)),
            pl.BlockSpec((bd, d_model), lambda j, i: (j, 0)),
            pl.BlockSpec((1, d_dict), lambda j, i: (0, 0)),
        ],
        out_specs=pl.BlockSpec((tmz, bd), lambda j, i: (i, j)),
        out_shape=jax.ShapeDtypeStruct((n_tok, d_dict), jnp.float32),
        compiler_params=pltpu.CompilerParams(
            dimension_semantics=("parallel", "parallel")),
        interpret=interpret,
    )(x, W_enc, b_enc2)

    tau = _make_sc_t0(n_tok, d_dict, topk, n_tok // 32,
                      interpret=interpret)(z)

    ni2, nk = n_tok // tm2, d_dict // kd
    b_dec2 = b_dec.reshape(1, d_model)
    x_hat, sparse = pl.pallas_call(
        functools.partial(_decode_fused_body, nk=nk),
        grid=(ni2, nk),
        in_specs=[
            pl.BlockSpec((tm2, kd), lambda i, k: (i, k)),
            pl.BlockSpec((tm2, 1), lambda i, k: (i, 0)),
            pl.BlockSpec((d_model, kd), lambda i, k: (0, k)),
            pl.BlockSpec((1, d_model), lambda i, k: (0, 0)),
        ],
        out_specs=[
            pl.BlockSpec((tm2, d_model), lambda i, k: (i, 0)),
            pl.BlockSpec((tm2, kd), lambda i, k: (i, k)),
        ],
        out_shape=[
            jax.ShapeDtypeStruct((n_tok, d_model), jnp.float32),
            jax.ShapeDtypeStruct((n_tok, d_dict), jnp.float32),
        ],
        compiler_params=pltpu.CompilerParams(
            dimension_semantics=("parallel", "arbitrary")),
        interpret=interpret,
    )(z, tau.reshape(n_tok, 1), W_dec, b_dec2)
    return (x_hat, sparse)


def kernel(x, W_enc, b_enc, W_dec, b_dec):
    return _topksae_fwd(x, W_enc, b_enc, W_dec, b_dec,
                        topk=64, tmz=512, bd=2048, tm2=512, kd=1024)
